# baseline (speedup 1.0000x reference)
import functools

import jax
import jax.numpy as jnp
from jax import lax
from jax.experimental import pallas as pl
from jax.experimental.pallas import tpu as pltpu
from jax.experimental.pallas import tpu_sc as plsc

LANES = 16
R = 4
FC = 1024


def _pack_body(lgt_ref, l_ref, r_ref, lr_ref, ab_ref, cd_ref):
    om = 0.5 + 0.5 * jnp.sin(lgt_ref[...])
    w00 = om[0:1]
    w01 = om[1:2]
    w10 = om[2:3]
    w11 = om[3:4]
    a = w00
    b = w10 - w00
    c = w01 - w00
    d = (w00 - w01) + (w11 - w10)
    lr_ref[...] = l_ref[...] | (r_ref[...] << 16)

    def pack_pair(lo, hi):
        lo16 = lax.bitcast_convert_type(lo.astype(jnp.bfloat16), jnp.uint16)
        hi16 = lax.bitcast_convert_type(hi.astype(jnp.bfloat16), jnp.uint16)
        return (lo16.astype(jnp.int32) | (hi16.astype(jnp.int32) << 16))

    ab_ref[...] = pack_pair(a, b)
    cd_ref[...] = pack_pair(c, d)


def _pack_metadata(logits, left, right):
    out_f = left.shape[0]
    lgt = logits.T
    shp = jax.ShapeDtypeStruct((1, out_f), jnp.int32)
    lr, ab, cd = pl.pallas_call(
        _pack_body,
        out_shape=(shp, shp, shp),
    )(lgt, left.reshape(1, out_f), right.reshape(1, out_f))
    return lr.reshape(-1), ab.reshape(-1), cd.reshape(-1)


@functools.partial(jax.jit, static_argnames=("batch", "in_f", "out_f"))
def _run_sc(x_flat, lr, ab, cd, *, batch, in_f, out_f):
    info = plsc.get_sparse_core_info()
    nc, ns = info.num_cores, info.num_subcores
    nw = nc * ns
    rows_per = batch // nw
    n_gp = rows_per // (2 * R)
    n_chunks = out_f // FC
    jsteps = FC // LANES
    assert rows_per == n_gp * 2 * R and out_f == n_chunks * FC

    mesh = plsc.VectorSubcoreMesh(core_axis_name="c", subcore_axis_name="s")

    @functools.partial(
        pl.kernel,
        mesh=mesh,
        compiler_params=pltpu.CompilerParams(needs_layout_passes=False),
        out_type=jax.ShapeDtypeStruct((batch, out_f), jnp.float32),
        scratch_types=[
            pltpu.VMEM((out_f,), jnp.int32),
            pltpu.VMEM((out_f,), jnp.int32),
            pltpu.VMEM((out_f,), jnp.int32),
        ]
        + [pltpu.VMEM((in_f,), jnp.float32) for _ in range(2 * R)]
        + [pltpu.VMEM((R, FC), jnp.float32) for _ in range(2)]
        + [pltpu.SemaphoreType.DMA for _ in range(4)],
    )
    def sc_kernel(x_hbm, lr_hbm, ab_hbm, cd_hbm, out_hbm,
                  lr_v, ab_v, cd_v,
                  x00, x01, x02, x03, x10, x11, x12, x13,
                  ob0, ob1, sx0, sx1, so0, so1):
        xbufs = ((x00, x01, x02, x03), (x10, x11, x12, x13))
        obufs = (ob0, ob1)
        sxs = (sx0, sx1)
        sos = (so0, so1)
        wid = lax.axis_index("s") * nc + lax.axis_index("c")
        row_base = wid * rows_per

        pltpu.sync_copy(lr_hbm, lr_v)
        pltpu.sync_copy(ab_hbm, ab_v)
        pltpu.sync_copy(cd_hbm, cd_v)

        def fetch_group(row0, bufs, sem):
            for r in range(R):
                pltpu.async_copy(
                    x_hbm.at[pl.ds((row0 + r) * in_f, in_f)], bufs[r], sem)

        def wait_group(bufs, sem):
            for r in range(R):
                pltpu.make_async_copy(
                    x_hbm.at[pl.ds(0, in_f)], bufs[r], sem).wait()

        def wait_out(q, row0):
            pltpu.make_async_copy(
                obufs[q], out_hbm.at[pl.ds(row0, R), pl.ds(0, FC)],
                sos[q]).wait()

        fetch_group(row_base, xbufs[0], sxs[0])

        def gp_body(gp, _):
            for p in (0, 1):
                g = gp * 2 + p
                row0 = row_base + g * R
                if p == 0:
                    fetch_group(row0 + R, xbufs[1], sxs[1])
                else:
                    @pl.when(gp < n_gp - 1)
                    def _prefetch():
                        fetch_group(row0 + R, xbufs[0], sxs[0])
                wait_group(xbufs[p], sxs[p])

                for c in range(n_chunks):
                    q = c % 2
                    ob = obufs[q]
                    if c >= 2 or p == 1:
                        wait_out(q, row0)
                    else:
                        @pl.when(gp > 0)
                        def _wait_prev():
                            wait_out(q, row0)

                    def j_body(jj, _, ob=ob, c=c, p=p):
                        base = c * FC + jj * LANES
                        vlr = lr_v[pl.ds(base, LANES)]
                        il = vlr & 0xFFFF
                        ir = lax.shift_right_logical(vlr, 16)
                        vab = ab_v[pl.ds(base, LANES)]
                        vcd = cd_v[pl.ds(base, LANES)]
                        ca = plsc.bitcast(vab << 16, jnp.float32)
                        cb = plsc.bitcast(vab & jnp.int32(-65536), jnp.float32)
                        cc = plsc.bitcast(vcd << 16, jnp.float32)
                        cd_ = plsc.bitcast(vcd & jnp.int32(-65536), jnp.float32)
                        for r in range(R):
                            lv = plsc.load_gather(xbufs[p][r], [il])
                            rv = plsc.load_gather(xbufs[p][r], [ir])
                            ob[r, pl.ds(jj * LANES, LANES)] = (
                                (ca + cb * lv) + (cc + cd_ * lv) * rv)
                        return _

                    lax.fori_loop(0, jsteps, j_body, None)
                    pltpu.async_copy(
                        ob, out_hbm.at[pl.ds(row0, R), pl.ds(c * FC, FC)],
                        sos[q])
            return _

        lax.fori_loop(0, n_gp, gp_body, None)
        for q in (0, 1):
            wait_out(q, row_base)

    return sc_kernel(x_flat, lr, ab, cd)


def kernel(x, logits, left_indices, right_indices):
    batch, in_f = x.shape
    out_f = left_indices.shape[0]
    lr, ab, cd = _pack_metadata(logits, left_indices, right_indices)
    return _run_sc(x.reshape(-1), lr, ab, cd,
                   batch=batch, in_f=in_f, out_f=out_f)


# speedup vs baseline: 3.2706x; 3.2706x over previous
import functools

import jax
import jax.numpy as jnp
from jax import lax
from jax.experimental import pallas as pl
from jax.experimental.pallas import tpu as pltpu
from jax.experimental.pallas import tpu_sc as plsc

LANES = 16
R = 4
FC = 1024


def _pack_body(lgt_ref, l_ref, r_ref, lr_ref, ab_ref, cd_ref):
    om = 0.5 + 0.5 * jnp.sin(lgt_ref[...])
    w00 = om[0:1]
    w01 = om[1:2]
    w10 = om[2:3]
    w11 = om[3:4]
    a = w00
    b = w10 - w00
    c = w01 - w00
    d = (w00 - w01) + (w11 - w10)
    lr_ref[...] = l_ref[...] | (r_ref[...] << 16)

    def pack_pair(lo, hi):
        lo16 = lax.bitcast_convert_type(lo.astype(jnp.bfloat16), jnp.uint16)
        hi16 = lax.bitcast_convert_type(hi.astype(jnp.bfloat16), jnp.uint16)
        return (lo16.astype(jnp.int32) | (hi16.astype(jnp.int32) << 16))

    ab_ref[...] = pack_pair(a, b)
    cd_ref[...] = pack_pair(c, d)


def _pack_metadata(logits, left, right):
    out_f = left.shape[0]
    lgt = logits.T
    shp = jax.ShapeDtypeStruct((1, out_f), jnp.int32)
    lr, ab, cd = pl.pallas_call(
        _pack_body,
        out_shape=(shp, shp, shp),
    )(lgt, left.reshape(1, out_f), right.reshape(1, out_f))
    return lr.reshape(-1), ab.reshape(-1), cd.reshape(-1)


@functools.partial(jax.jit, static_argnames=("batch", "in_f", "out_f"))
def _run_sc(x_flat, lr, ab, cd, *, batch, in_f, out_f):
    info = plsc.get_sparse_core_info()
    nc, ns = info.num_cores, info.num_subcores
    nw = nc * ns
    rows_per = batch // nw
    n_gp = rows_per // (2 * R)
    n_chunks = out_f // FC
    jsteps = FC // LANES
    assert rows_per == n_gp * 2 * R and out_f == n_chunks * FC

    mesh = plsc.VectorSubcoreMesh(core_axis_name="c", subcore_axis_name="s")

    @functools.partial(
        pl.kernel,
        mesh=mesh,
        compiler_params=pltpu.CompilerParams(
            needs_layout_passes=False, disable_bounds_checks=True),
        out_type=jax.ShapeDtypeStruct((batch, out_f), jnp.float32),
        scratch_types=[
            pltpu.VMEM((out_f,), jnp.int32),
            pltpu.VMEM((out_f,), jnp.int32),
            pltpu.VMEM((out_f,), jnp.int32),
        ]
        + [pltpu.VMEM((in_f,), jnp.float32) for _ in range(2 * R)]
        + [pltpu.VMEM((R, FC), jnp.float32) for _ in range(2)]
        + [pltpu.SemaphoreType.DMA for _ in range(4)],
    )
    def sc_kernel(x_hbm, lr_hbm, ab_hbm, cd_hbm, out_hbm,
                  lr_v, ab_v, cd_v,
                  x00, x01, x02, x03, x10, x11, x12, x13,
                  ob0, ob1, sx0, sx1, so0, so1):
        xbufs = ((x00, x01, x02, x03), (x10, x11, x12, x13))
        obufs = (ob0, ob1)
        sxs = (sx0, sx1)
        sos = (so0, so1)
        wid = lax.axis_index("s") * nc + lax.axis_index("c")
        row_base = wid * rows_per

        pltpu.sync_copy(lr_hbm, lr_v)
        pltpu.sync_copy(ab_hbm, ab_v)
        pltpu.sync_copy(cd_hbm, cd_v)

        def fetch_group(row0, bufs, sem):
            for r in range(R):
                pltpu.async_copy(
                    x_hbm.at[pl.ds((row0 + r) * in_f, in_f)], bufs[r], sem)

        def wait_group(bufs, sem):
            for r in range(R):
                pltpu.make_async_copy(
                    x_hbm.at[pl.ds(0, in_f)], bufs[r], sem).wait()

        def wait_out(q, row0):
            pltpu.make_async_copy(
                obufs[q], out_hbm.at[pl.ds(row0, R), pl.ds(0, FC)],
                sos[q]).wait()

        fetch_group(row_base, xbufs[0], sxs[0])

        def gp_body(gp, _):
            for p in (0, 1):
                g = gp * 2 + p
                row0 = row_base + g * R
                if p == 0:
                    fetch_group(row0 + R, xbufs[1], sxs[1])
                else:
                    @pl.when(gp < n_gp - 1)
                    def _prefetch():
                        fetch_group(row0 + R, xbufs[0], sxs[0])
                wait_group(xbufs[p], sxs[p])

                for c in range(n_chunks):
                    q = c % 2
                    ob = obufs[q]
                    if c >= 2 or p == 1:
                        wait_out(q, row0)
                    else:
                        @pl.when(gp > 0)
                        def _wait_prev():
                            wait_out(q, row0)

                    @plsc.parallel_loop(0, jsteps, step=1, unroll=4)
                    def j_body(jj, ob=ob, c=c, p=p):
                        base = c * FC + jj * LANES
                        vlr = lr_v[pl.ds(base, LANES)]
                        il = vlr & 0xFFFF
                        ir = lax.shift_right_logical(vlr, 16)
                        vab = ab_v[pl.ds(base, LANES)]
                        vcd = cd_v[pl.ds(base, LANES)]
                        ca = plsc.bitcast(vab << 16, jnp.float32)
                        cb = plsc.bitcast(vab & jnp.int32(-65536), jnp.float32)
                        cc = plsc.bitcast(vcd << 16, jnp.float32)
                        cd_ = plsc.bitcast(vcd & jnp.int32(-65536), jnp.float32)
                        for r in range(R):
                            lv = plsc.load_gather(xbufs[p][r], [il])
                            rv = plsc.load_gather(xbufs[p][r], [ir])
                            ob[r, pl.ds(jj * LANES, LANES)] = (
                                (ca + cb * lv) + (cc + cd_ * lv) * rv)
                    pltpu.async_copy(
                        ob, out_hbm.at[pl.ds(row0, R), pl.ds(c * FC, FC)],
                        sos[q])
            return _

        lax.fori_loop(0, n_gp, gp_body, None)
        for q in (0, 1):
            wait_out(q, row_base)

    return sc_kernel(x_flat, lr, ab, cd)


def kernel(x, logits, left_indices, right_indices):
    batch, in_f = x.shape
    out_f = left_indices.shape[0]
    lr, ab, cd = _pack_metadata(logits, left_indices, right_indices)
    return _run_sc(x.reshape(-1), lr, ab, cd,
                   batch=batch, in_f=in_f, out_f=out_f)


# speedup vs baseline: 4.1613x; 1.2723x over previous
import functools

import jax
import jax.numpy as jnp
from jax import lax
from jax.experimental import pallas as pl
from jax.experimental.pallas import tpu as pltpu
from jax.experimental.pallas import tpu_sc as plsc

LANES = 16
R = 4
FC = 1024


def _pack_body(lgt_ref, l_ref, r_ref, lr_ref, ab_ref, cd_ref):
    om = 0.5 + 0.5 * jnp.sin(lgt_ref[...])
    w00 = om[0:1]
    w01 = om[1:2]
    w10 = om[2:3]
    w11 = om[3:4]
    a = w00
    b = w10 - w00
    c = w01 - w00
    d = (w00 - w01) + (w11 - w10)
    lr_ref[...] = l_ref[...] | (r_ref[...] << 16)

    def pack_pair(lo, hi):
        lo16 = lax.bitcast_convert_type(lo.astype(jnp.bfloat16), jnp.uint16)
        hi16 = lax.bitcast_convert_type(hi.astype(jnp.bfloat16), jnp.uint16)
        return (lo16.astype(jnp.int32) | (hi16.astype(jnp.int32) << 16))

    ab_ref[...] = pack_pair(a, b)
    cd_ref[...] = pack_pair(c, d)


def _pack_metadata(logits, left, right):
    out_f = left.shape[0]
    lgt = logits.T
    shp = jax.ShapeDtypeStruct((1, out_f), jnp.int32)
    lr, ab, cd = pl.pallas_call(
        _pack_body,
        out_shape=(shp, shp, shp),
    )(lgt, left.reshape(1, out_f), right.reshape(1, out_f))
    return lr.reshape(-1), ab.reshape(-1), cd.reshape(-1)


@functools.partial(jax.jit, static_argnames=("batch", "in_f", "out_f"))
def _run_sc(x2d, lr, ab, cd, *, batch, in_f, out_f):
    info = plsc.get_sparse_core_info()
    nc, ns = info.num_cores, info.num_subcores
    nw = nc * ns
    rows_per = batch // nw
    n_gp = rows_per // (2 * R)
    n_chunks = out_f // FC
    jsteps = FC // LANES
    assert rows_per == n_gp * 2 * R and out_f == n_chunks * FC

    mesh = plsc.VectorSubcoreMesh(core_axis_name="c", subcore_axis_name="s")

    @functools.partial(
        pl.kernel,
        mesh=mesh,
        compiler_params=pltpu.CompilerParams(
            needs_layout_passes=False, disable_bounds_checks=True),
        out_type=jax.ShapeDtypeStruct((batch, out_f), jnp.float32),
        scratch_types=[
            pltpu.VMEM((out_f,), jnp.int32),
            pltpu.VMEM((out_f,), jnp.int32),
            pltpu.VMEM((out_f,), jnp.int32),
        ]
        + [pltpu.VMEM((in_f,), jnp.float32) for _ in range(2 * R)]
        + [pltpu.VMEM((R, FC), jnp.float32) for _ in range(2)]
        + [pltpu.SemaphoreType.DMA for _ in range(4)],
    )
    def sc_kernel(x_hbm, lr_hbm, ab_hbm, cd_hbm, out_hbm,
                  lr_v, ab_v, cd_v,
                  x00, x01, x02, x03, x10, x11, x12, x13,
                  ob0, ob1, sx0, sx1, so0, so1):
        xbufs = ((x00, x01, x02, x03), (x10, x11, x12, x13))
        obufs = (ob0, ob1)
        sxs = (sx0, sx1)
        sos = (so0, so1)
        wid = lax.axis_index("s") * nc + lax.axis_index("c")
        row_base = wid * rows_per

        pltpu.sync_copy(lr_hbm, lr_v)
        pltpu.sync_copy(ab_hbm, ab_v)
        pltpu.sync_copy(cd_hbm, cd_v)

        def fetch_group(row0, bufs, sem):
            for r in range(R):
                pltpu.async_copy(x_hbm.at[row0 + r], bufs[r], sem)

        def wait_group(bufs, sem):
            for r in range(R):
                pltpu.make_async_copy(x_hbm.at[0], bufs[r], sem).wait()

        def wait_out(q, row0):
            pltpu.make_async_copy(
                obufs[q], out_hbm.at[pl.ds(row0, R), pl.ds(0, FC)],
                sos[q]).wait()

        fetch_group(row_base, xbufs[0], sxs[0])

        def gp_body(gp, _):
            for p in (0, 1):
                g = gp * 2 + p
                row0 = row_base + g * R
                if p == 0:
                    fetch_group(row0 + R, xbufs[1], sxs[1])
                else:
                    @pl.when(gp < n_gp - 1)
                    def _prefetch():
                        fetch_group(row0 + R, xbufs[0], sxs[0])
                wait_group(xbufs[p], sxs[p])

                for c in range(n_chunks):
                    q = c % 2
                    ob = obufs[q]
                    if c >= 2 or p == 1:
                        wait_out(q, row0)
                    else:
                        @pl.when(gp > 0)
                        def _wait_prev():
                            wait_out(q, row0)

                    @plsc.parallel_loop(0, jsteps, step=1, unroll=4)
                    def j_body(jj, ob=ob, c=c, p=p):
                        base = c * FC + jj * LANES
                        vlr = lr_v[pl.ds(base, LANES)]
                        il = vlr & 0xFFFF
                        ir = lax.shift_right_logical(vlr, 16)
                        vab = ab_v[pl.ds(base, LANES)]
                        vcd = cd_v[pl.ds(base, LANES)]
                        ca = plsc.bitcast(vab << 16, jnp.float32)
                        cb = plsc.bitcast(vab & jnp.int32(-65536), jnp.float32)
                        cc = plsc.bitcast(vcd << 16, jnp.float32)
                        cd_ = plsc.bitcast(vcd & jnp.int32(-65536), jnp.float32)
                        for r in range(R):
                            lv = plsc.load_gather(xbufs[p][r], [il])
                            rv = plsc.load_gather(xbufs[p][r], [ir])
                            ob[r, pl.ds(jj * LANES, LANES)] = (
                                (ca + cb * lv) + (cc + cd_ * lv) * rv)
                    pltpu.async_copy(
                        ob, out_hbm.at[pl.ds(row0, R), pl.ds(c * FC, FC)],
                        sos[q])
            return _

        lax.fori_loop(0, n_gp, gp_body, None)
        for q in (0, 1):
            wait_out(q, row_base)

    return sc_kernel(x2d, lr, ab, cd)


def kernel(x, logits, left_indices, right_indices):
    batch, in_f = x.shape
    out_f = left_indices.shape[0]
    lr, ab, cd = _pack_metadata(logits, left_indices, right_indices)
    return _run_sc(x, lr, ab, cd,
                   batch=batch, in_f=in_f, out_f=out_f)


# speedup vs baseline: 4.6652x; 1.1211x over previous
import functools

import jax
import jax.numpy as jnp
from jax import lax
from jax.experimental import pallas as pl
from jax.experimental.pallas import tpu as pltpu
from jax.experimental.pallas import tpu_sc as plsc

LANES = 16
P = 4
FC = 256
XBLK = 128


def _pack_body(lgt_ref, l_ref, r_ref, lr_ref, ab_ref, cd_ref):
    om = 0.5 + 0.5 * jnp.sin(lgt_ref[...])
    w00 = om[0:1]
    w01 = om[1:2]
    w10 = om[2:3]
    w11 = om[3:4]
    a = w00
    b = w10 - w00
    c = w01 - w00
    d = (w00 - w01) + (w11 - w10)
    lr_ref[...] = l_ref[...] | (r_ref[...] << 16)

    def pack_pair(lo, hi):
        lo16 = lax.bitcast_convert_type(lo.astype(jnp.bfloat16), jnp.uint16)
        hi16 = lax.bitcast_convert_type(hi.astype(jnp.bfloat16), jnp.uint16)
        return (lo16.astype(jnp.int32) | (hi16.astype(jnp.int32) << 16))

    ab_ref[...] = pack_pair(a, b)
    cd_ref[...] = pack_pair(c, d)


def _pack_metadata(logits, left, right):
    out_f = left.shape[0]
    lgt = logits.T
    shp = jax.ShapeDtypeStruct((1, out_f), jnp.int32)
    lr, ab, cd = pl.pallas_call(
        _pack_body,
        out_shape=(shp, shp, shp),
    )(lgt, left.reshape(1, out_f), right.reshape(1, out_f))
    return lr.reshape(-1), ab.reshape(-1), cd.reshape(-1)


def _pack_x_body(lo_ref, hi_ref, xp_ref):
    lo = lax.bitcast_convert_type(
        lo_ref[...].astype(jnp.bfloat16), jnp.uint16)
    hi = lax.bitcast_convert_type(
        hi_ref[...].astype(jnp.bfloat16), jnp.uint16)
    xp_ref[...] = (lo.astype(jnp.int32) | (hi.astype(jnp.int32) << 16))


def _pack_x(x):
    batch, in_f = x.shape
    grid = batch // 2 // XBLK
    return pl.pallas_call(
        _pack_x_body,
        grid=(grid,),
        in_specs=[
            pl.BlockSpec((XBLK, in_f), lambda i: (i, 0)),
            pl.BlockSpec((XBLK, in_f), lambda i: (i + grid, 0)),
        ],
        out_specs=pl.BlockSpec((XBLK, in_f), lambda i: (i, 0)),
        out_shape=jax.ShapeDtypeStruct((batch // 2, in_f), jnp.int32),
    )(x, x)


@functools.partial(jax.jit, static_argnames=("batch", "in_f", "out_f"))
def _run_sc(xp, lr, ab, cd, *, batch, in_f, out_f):
    info = plsc.get_sparse_core_info()
    nc, ns = info.num_cores, info.num_subcores
    nw = nc * ns
    pairs_per = (batch // 2) // nw
    n_gp = pairs_per // (2 * P)
    n_chunks = out_f // FC
    jsteps = FC // LANES
    assert pairs_per == n_gp * 2 * P and out_f == n_chunks * FC

    mesh = plsc.VectorSubcoreMesh(core_axis_name="c", subcore_axis_name="s")

    @functools.partial(
        pl.kernel,
        mesh=mesh,
        compiler_params=pltpu.CompilerParams(
            needs_layout_passes=False, disable_bounds_checks=True),
        out_type=jax.ShapeDtypeStruct((batch, out_f), jnp.float32),
        scratch_types=[
            pltpu.VMEM((out_f,), jnp.int32),
            pltpu.VMEM((out_f,), jnp.int32),
            pltpu.VMEM((out_f,), jnp.int32),
        ]
        + [pltpu.VMEM((in_f,), jnp.int32) for _ in range(2 * P)]
        + [pltpu.VMEM((2 * P, FC), jnp.float32) for _ in range(2)]
        + [pltpu.SemaphoreType.DMA for _ in range(4)],
    )
    def sc_kernel(xp_hbm, lr_hbm, ab_hbm, cd_hbm, out_hbm,
                  lr_v, ab_v, cd_v,
                  x00, x01, x02, x03, x10, x11, x12, x13,
                  ob0, ob1, sx0, sx1, so0, so1):
        xbufs = ((x00, x01, x02, x03), (x10, x11, x12, x13))
        obufs = (ob0, ob1)
        sxs = (sx0, sx1)
        sos = (so0, so1)
        wid = lax.axis_index("s") * nc + lax.axis_index("c")
        pair_base = wid * pairs_per

        pltpu.sync_copy(lr_hbm, lr_v)
        pltpu.sync_copy(ab_hbm, ab_v)
        pltpu.sync_copy(cd_hbm, cd_v)

        def fetch_group(pair0, bufs, sem):
            for q in range(P):
                pltpu.async_copy(xp_hbm.at[pair0 + q], bufs[q], sem)

        def wait_group(bufs, sem):
            for q in range(P):
                pltpu.make_async_copy(xp_hbm.at[0], bufs[q], sem).wait()

        half = batch // 2

        def send_out(q, pair0, c):
            pltpu.async_copy(
                obufs[q].at[pl.ds(0, P), :],
                out_hbm.at[pl.ds(pair0, P), pl.ds(c * FC, FC)], sos[q])
            pltpu.async_copy(
                obufs[q].at[pl.ds(P, P), :],
                out_hbm.at[pl.ds(half + pair0, P), pl.ds(c * FC, FC)], sos[q])

        def wait_out(q, pair0):
            for h in (0, 1):
                pltpu.make_async_copy(
                    obufs[q].at[pl.ds(0, P), :],
                    out_hbm.at[pl.ds(pair0, P), pl.ds(0, FC)],
                    sos[q]).wait()

        fetch_group(pair_base, xbufs[0], sxs[0])

        def gp_body(gp, _):
            for hp in (0, 1):
                g = gp * 2 + hp
                pair0 = pair_base + g * P
                if hp == 0:
                    fetch_group(pair0 + P, xbufs[1], sxs[1])
                else:
                    @pl.when(gp < n_gp - 1)
                    def _prefetch():
                        fetch_group(pair0 + P, xbufs[0], sxs[0])
                wait_group(xbufs[hp], sxs[hp])

                def chunk_body(ci, _, hp=hp, pair0=pair0):
                    for par in (0, 1):
                        c = ci * 2 + par
                        ob = obufs[par]
                        if hp == 1:
                            wait_out(par, pair0)
                        else:
                            @pl.when((gp > 0) | (c >= 2))
                            def _wait_prev():
                                wait_out(par, pair0)

                        @plsc.parallel_loop(0, jsteps, step=1, unroll=2)
                        def j_body(jj, ob=ob, c=c, hp=hp):
                            base = c * FC + jj * LANES
                            vlr = lr_v[pl.ds(base, LANES)]
                            il = vlr & 0xFFFF
                            ir = lax.shift_right_logical(vlr, 16)
                            vab = ab_v[pl.ds(base, LANES)]
                            vcd = cd_v[pl.ds(base, LANES)]
                            ca = plsc.bitcast((vab & 0xFFFF) | (vab << 16),
                                              jnp.bfloat16)
                            cb = plsc.bitcast(
                                lax.shift_right_logical(vab, 16)
                                | (vab & jnp.int32(-65536)), jnp.bfloat16)
                            cc = plsc.bitcast((vcd & 0xFFFF) | (vcd << 16),
                                              jnp.bfloat16)
                            cdv = plsc.bitcast(
                                lax.shift_right_logical(vcd, 16)
                                | (vcd & jnp.int32(-65536)), jnp.bfloat16)
                            for q in range(P):
                                lv = plsc.bitcast(
                                    plsc.load_gather(xbufs[hp][q], [il]),
                                    jnp.bfloat16)
                                rv = plsc.bitcast(
                                    plsc.load_gather(xbufs[hp][q], [ir]),
                                    jnp.bfloat16)
                                res = (ca + cb * lv) + (cc + cdv * lv) * rv
                                ev, od = plsc.unpack(
                                    res, format=plsc.PackFormat.INTERLEAVED)
                                ob[q, pl.ds(jj * LANES, LANES)] = ev
                                ob[P + q, pl.ds(jj * LANES, LANES)] = od
                        send_out(par, pair0, c)
                    return _
                lax.fori_loop(0, n_chunks // 2, chunk_body, None)
            return _

        lax.fori_loop(0, n_gp, gp_body, None)
        for q in (0, 1):
            wait_out(q, pair_base)

    return sc_kernel(xp, lr, ab, cd)


def kernel(x, logits, left_indices, right_indices):
    batch, in_f = x.shape
    out_f = left_indices.shape[0]
    lr, ab, cd = _pack_metadata(logits, left_indices, right_indices)
    xp = _pack_x(x)
    return _run_sc(xp, lr, ab, cd,
                   batch=batch, in_f=in_f, out_f=out_f)
